# Initial kernel scaffold; baseline (speedup 1.0000x reference)
import functools

import jax
import jax.numpy as jnp
from jax import lax
from jax.experimental import pallas as pl
from jax.experimental.pallas import tpu as pltpu
from jax.experimental.pallas import tpu_sc as plsc

N = 50000
E = 800000
D = 100

NC = 2
NS = 16
L = 16
DP = 112
NB = 4
BR = N // NB
BPC = NB // NC
R = 5000
EPT = E // NS
NR = EPT // R
K = 128
ACC_STRIPE = 784
ACC_ROWS = ACC_STRIPE * NS
DUMMY_ROW = BR


def _layer_body(row_hbm, col_hbm, w_hbm, table_hbm, out_hbm,
                e_row, e_col, e_w, b_col, b_w, b_rl, idx_rl,
                gbuf, zbuf, acc, sem):
  c = lax.axis_index("c")
  s = lax.axis_index("s")
  ebase = s * EPT

  def zrow(r, _):
    for q in range(DP // L):
      zbuf[r, pl.ds(q * L, L)] = jnp.zeros((L,), jnp.float32)
    return 0
  lax.fori_loop(0, zbuf.shape[0], zrow, 0)

  ziota = lax.iota(jnp.int32, L)

  for b in range(BPC):
    lo = (c * BPC + b) * BR

    for q in range(ACC_STRIPE // zbuf.shape[0]):
      pltpu.sync_copy(
          zbuf, acc.at[pl.ds(s * ACC_STRIPE + q * zbuf.shape[0],
                             zbuf.shape[0])])
    plsc.subcore_barrier()

    def round_body(r, _):
      off = ebase + r * R
      pltpu.sync_copy(row_hbm.at[pl.ds(off, R)], e_row)
      pltpu.sync_copy(col_hbm.at[pl.ds(off, R)], e_col)
      pltpu.sync_copy(w_hbm.at[pl.ds(off, R)], e_w)

      def comp(i, cnt):
        rows = e_row[pl.ds(i * L, L)]
        cols = e_col[pl.ds(i * L, L)]
        ws = e_w[pl.ds(i * L, L)]
        m = (rows >= lo) & (rows < lo + BR)
        mi = m.astype(jnp.int32)
        pos = cnt + plsc.cumsum(mi) - 1
        plsc.store_scatter(b_col, [pos], cols, mask=m)
        plsc.store_scatter(b_w, [pos], ws, mask=m)
        plsc.store_scatter(b_rl, [pos], rows - lo, mask=m)
        return cnt + jnp.sum(mi)
      cnt = lax.fori_loop(0, R // L, comp, jnp.int32(0))

      for q in range(K // L):
        padpos = cnt + q * L + ziota
        plsc.store_scatter(b_col, [padpos], jnp.zeros((L,), jnp.int32))
        plsc.store_scatter(b_w, [padpos], jnp.zeros((L,), jnp.float32))
        plsc.store_scatter(b_rl, [padpos],
                           jnp.full((L,), DUMMY_ROW, jnp.int32))
      nch = (cnt + (K - 1)) // K

      def chunk(j, _):
        koff = j * K
        pltpu.async_copy(
            table_hbm.at[b_col.at[pl.ds(koff, K)]], gbuf, sem).wait()
        for q in range(K // L):
          idx_rl[pl.ds(q * L, L)] = b_rl[pl.ds(koff + q * L, L)]

        def scale(e2, _):
          wv = plsc.load_gather(
              b_w, [jnp.zeros((L,), jnp.int32) + (koff + e2)])
          for q in range(DP // L):
            gbuf[e2, pl.ds(q * L, L)] = gbuf[e2, pl.ds(q * L, L)] * wv
          return 0
        lax.fori_loop(0, K, scale, 0)

        pltpu.sync_copy(gbuf, acc.at[idx_rl], add=True)
        return 0
      lax.fori_loop(0, nch, chunk, 0)
      return 0
    lax.fori_loop(0, NR, round_body, 0)
    plsc.subcore_barrier()

    last = BR - (NS - 1) * ACC_STRIPE

    @pl.when(s < NS - 1)
    def _():
      pltpu.sync_copy(acc.at[pl.ds(s * ACC_STRIPE, ACC_STRIPE)],
                      out_hbm.at[pl.ds(lo + s * ACC_STRIPE, ACC_STRIPE)])

    @pl.when(s == NS - 1)
    def _():
      pltpu.sync_copy(acc.at[pl.ds((NS - 1) * ACC_STRIPE, last)],
                      out_hbm.at[pl.ds(lo + (NS - 1) * ACC_STRIPE, last)])


_sc_layer = pl.kernel(
    _layer_body,
    out_type=jax.ShapeDtypeStruct((N, DP), jnp.float32),
    mesh=plsc.VectorSubcoreMesh(core_axis_name="c", subcore_axis_name="s"),
    scratch_types=[
        pltpu.VMEM((R,), jnp.int32),
        pltpu.VMEM((R,), jnp.int32),
        pltpu.VMEM((R,), jnp.float32),
        pltpu.VMEM((R + 2 * K,), jnp.int32),
        pltpu.VMEM((R + 2 * K,), jnp.float32),
        pltpu.VMEM((R + 2 * K,), jnp.int32),
        pltpu.VMEM((K,), jnp.int32),
        pltpu.VMEM((K, DP), jnp.float32),
        pltpu.VMEM((DP, DP), jnp.float32),
        pltpu.VMEM_SHARED((ACC_ROWS, DP), jnp.float32),
        pltpu.SemaphoreType.DMA,
    ],
)


_CROWS = 1000


def _combine_body(a_ref, h0, h1, h2, h3, o_ref):
  acc = jnp.zeros((_CROWS, DP), jnp.float32)
  for l, h in enumerate((h0, h1, h2, h3)):
    x = h[...]
    ss = jnp.sum(x * x, axis=-1, keepdims=True)
    nrm = jnp.maximum(jnp.sqrt(ss), 1e-12)
    acc = acc + a_ref[l] * (x / nrm)
  o_ref[...] = acc


_combine = pl.pallas_call(
    _combine_body,
    grid=(N // _CROWS,),
    in_specs=[
        pl.BlockSpec(memory_space=pltpu.SMEM),
    ] + [pl.BlockSpec((_CROWS, DP), lambda i: (i, 0)) for _ in range(4)],
    out_specs=pl.BlockSpec((_CROWS, DP), lambda i: (i, 0)),
    out_shape=jax.ShapeDtypeStruct((N, DP), jnp.float32),
)


def kernel(edge_index, edge_weight, embedding, a):
  row = edge_index[0]
  col = edge_index[1]
  x0 = jnp.pad(embedding, ((0, 0), (0, DP - D)))
  h1 = _sc_layer(row, col, edge_weight, x0)
  h2 = _sc_layer(row, col, edge_weight, h1)
  h3 = _sc_layer(row, col, edge_weight, h2)
  out = _combine(a.reshape(-1), x0, h1, h2, h3)
  return out[:, :D]


# speedup vs baseline: 4.3281x; 4.3281x over previous
import functools

import jax
import jax.numpy as jnp
from jax import lax
from jax.experimental import pallas as pl
from jax.experimental.pallas import tpu as pltpu
from jax.experimental.pallas import tpu_sc as plsc

N = 50000
E = 800000
D = 100

NC = 2
NS = 16
L = 16
DP = 112
NB = 4
BR = 12512
NP = NB * BR
BPC = NB // NC
R = 2000
EPT = E // NS
NR = EPT // R
K = 128
BCAP = R + 2 * K + 8
ACC_STRIPE = 784
ACC_ROWS = ACC_STRIPE * NS
DUMMY_ROW = BR
ZROWS = 16


def _layer_body(row_hbm, col_hbm, w_hbm, table_hbm, out_hbm,
                e_row, e_col, e_w, b_col, b_w, b_rl, idx_rl,
                gbuf, zbuf, acc, sem):
  c = lax.axis_index("c")
  s = lax.axis_index("s")
  ebase = s * EPT

  def zrow(r, _):
    for q in range(DP // L):
      zbuf[r, pl.ds(q * L, L)] = jnp.zeros((L,), jnp.float32)
    return 0
  lax.fori_loop(0, ZROWS, zrow, 0)

  ziota = lax.iota(jnp.int32, L)

  def chunk(j, _):
    koff = j * K
    pltpu.async_copy(
        table_hbm.at[b_col.at[pl.ds(koff, K)]], gbuf, sem).wait()
    for q in range(K // L):
      idx_rl[pl.ds(q * L, L)] = b_rl[pl.ds(koff + q * L, L)]

    def scale(e2, _):
      wv = plsc.load_gather(
          b_w, [jnp.zeros((L,), jnp.int32) + (koff + e2)])
      for q in range(DP // L):
        gbuf[e2, pl.ds(q * L, L)] = gbuf[e2, pl.ds(q * L, L)] * wv
      return 0
    lax.fori_loop(0, K, scale, 0)

    pltpu.sync_copy(gbuf, acc.at[idx_rl], add=True)
    return 0

  for b in range(BPC):
    lo = (c * BPC + b) * BR

    for q in range(ACC_STRIPE // ZROWS):
      pltpu.sync_copy(
          zbuf, acc.at[pl.ds(s * ACC_STRIPE + q * ZROWS, ZROWS)])
    plsc.subcore_barrier()

    def round_body(r, cnt):
      off = ebase + r * R
      pltpu.sync_copy(row_hbm.at[pl.ds(off, R)], e_row)
      pltpu.sync_copy(col_hbm.at[pl.ds(off, R)], e_col)
      pltpu.sync_copy(w_hbm.at[pl.ds(off, R)], e_w)

      def comp(i, cnt):
        rows = e_row[pl.ds(i * L, L)]
        cols = e_col[pl.ds(i * L, L)]
        ws = e_w[pl.ds(i * L, L)]
        m = (rows >= lo) & (rows < lo + BR)
        mi = jnp.where(m, jnp.ones((L,), jnp.int32),
                       jnp.zeros((L,), jnp.int32))
        pos = cnt + plsc.cumsum(mi) - 1
        plsc.store_scatter(b_col, [pos], cols, mask=m)
        plsc.store_scatter(b_w, [pos], ws, mask=m)
        plsc.store_scatter(b_rl, [pos], rows - lo, mask=m)
        return cnt + jnp.sum(mi)
      cnt = lax.fori_loop(0, R // L, comp, cnt)

      nch = cnt // K
      lax.fori_loop(0, nch, chunk, 0)
      rem_base = nch * K
      for q in range(K // L):
        b_col[pl.ds(q * L, L)] = b_col[pl.ds(rem_base + q * L, L)]
        b_w[pl.ds(q * L, L)] = b_w[pl.ds(rem_base + q * L, L)]
        b_rl[pl.ds(q * L, L)] = b_rl[pl.ds(rem_base + q * L, L)]
      return cnt - rem_base
    cnt = lax.fori_loop(0, NR, round_body, jnp.int32(0))

    for q in range(K // L):
      padpos = cnt + q * L + ziota
      plsc.store_scatter(b_col, [padpos], jnp.zeros((L,), jnp.int32))
      plsc.store_scatter(b_w, [padpos], jnp.zeros((L,), jnp.float32))
      plsc.store_scatter(b_rl, [padpos],
                         jnp.full((L,), DUMMY_ROW, jnp.int32))
    lax.fori_loop(0, (cnt + (K - 1)) // K, chunk, 0)
    plsc.subcore_barrier()

    last = BR - (NS - 1) * ACC_STRIPE

    @pl.when(s < NS - 1)
    def _():
      pltpu.sync_copy(acc.at[pl.ds(s * ACC_STRIPE, ACC_STRIPE)],
                      out_hbm.at[pl.ds(lo + s * ACC_STRIPE, ACC_STRIPE)])

    @pl.when(s == NS - 1)
    def _():
      pltpu.sync_copy(acc.at[pl.ds((NS - 1) * ACC_STRIPE, last)],
                      out_hbm.at[pl.ds(lo + (NS - 1) * ACC_STRIPE, last)])


_sc_layer = pl.kernel(
    _layer_body,
    out_type=jax.ShapeDtypeStruct((NP, DP), jnp.float32),
    mesh=plsc.VectorSubcoreMesh(core_axis_name="c", subcore_axis_name="s",
                                num_cores=NC, num_subcores=NS),
    compiler_params=pltpu.CompilerParams(needs_layout_passes=False,
                                         use_tc_tiling_on_sc=False),
    scratch_types=[
        pltpu.VMEM((R,), jnp.int32),
        pltpu.VMEM((R,), jnp.int32),
        pltpu.VMEM((R,), jnp.float32),
        pltpu.VMEM((BCAP,), jnp.int32),
        pltpu.VMEM((BCAP,), jnp.float32),
        pltpu.VMEM((BCAP,), jnp.int32),
        pltpu.VMEM((K,), jnp.int32),
        pltpu.VMEM((K, DP), jnp.float32),
        pltpu.VMEM((ZROWS, DP), jnp.float32),
        pltpu.VMEM_SHARED((ACC_ROWS, DP), jnp.float32),
        pltpu.SemaphoreType.DMA,
    ],
)


_CROWS = 3128


def _combine_body(a_ref, h0, h1, h2, h3, o_ref):
  acc = jnp.zeros((_CROWS, DP), jnp.float32)
  for l, h in enumerate((h0, h1, h2, h3)):
    x = h[...]
    ss = jnp.sum(x * x, axis=-1, keepdims=True)
    nrm = jnp.maximum(jnp.sqrt(ss), 1e-12)
    acc = acc + a_ref[l] * (x / nrm)
  o_ref[...] = acc


_combine = pl.pallas_call(
    _combine_body,
    grid=(NP // _CROWS,),
    in_specs=[
        pl.BlockSpec(memory_space=pltpu.SMEM),
    ] + [pl.BlockSpec((_CROWS, DP), lambda i: (i, 0)) for _ in range(4)],
    out_specs=pl.BlockSpec((_CROWS, DP), lambda i: (i, 0)),
    out_shape=jax.ShapeDtypeStruct((NP, DP), jnp.float32),
)


def kernel(edge_index, edge_weight, embedding, a):
  row = edge_index[0]
  col = edge_index[1]
  x0 = jnp.pad(embedding, ((0, NP - N), (0, DP - D)))
  h1 = _sc_layer(row, col, edge_weight, x0)
  h2 = _sc_layer(row, col, edge_weight, h1)
  h3 = _sc_layer(row, col, edge_weight, h2)
  out = _combine(a.reshape(-1), x0, h1, h2, h3)
  return out[:N, :D]


# speedup vs baseline: 5.3796x; 1.2429x over previous
import functools

import jax
import jax.numpy as jnp
from jax import lax
from jax.experimental import pallas as pl
from jax.experimental.pallas import tpu as pltpu
from jax.experimental.pallas import tpu_sc as plsc

N = 50000
E = 800000
D = 100

NC = 2
NS = 16
L = 16
DP = 112
NB = 4
BR = 12512
NP = NB * BR
BPC = NB // NC
R = 2000
EPT = E // NS
NR = EPT // R
K = 128
BCAP = R + 2 * K + 8
ACC_STRIPE = 784
ACC_ROWS = ACC_STRIPE * NS
DUMMY_ROW = BR


def _layer_body(row_hbm, col_hbm, w_hbm, table_hbm, out_hbm,
                e_row, e_col, e_w, b_col, b_w, b_rl, idx_rl,
                gbuf0, gbuf1, acc, sem0, sem1, sem2):
  c = lax.axis_index("c")
  s = lax.axis_index("s")
  ebase = s * EPT
  ziota = lax.iota(jnp.int32, L)
  gbufs = (gbuf0, gbuf1)
  sems = (sem0, sem1)

  def start_gather(j, p):
    pltpu.make_async_copy(
        table_hbm.at[b_col.at[pl.ds(j * K, K)]], gbufs[p], sems[p]).start()

  def wait_gather(p):
    pltpu.make_async_copy(
        table_hbm.at[b_col.at[pl.ds(0, K)]], gbufs[p], sems[p]).wait()

  def scale_scatter(j, p):
    gb = gbufs[p]
    koff = j * K
    for q in range(K // L):
      idx_rl[pl.ds(q * L, L)] = b_rl[pl.ds(koff + q * L, L)]

    def scale(e2, _):
      wv = plsc.load_gather(
          b_w, [jnp.zeros((L,), jnp.int32) + (koff + e2)])
      for q in range(DP // L):
        gb[e2, pl.ds(q * L, L)] = gb[e2, pl.ds(q * L, L)] * wv
      return 0
    lax.fori_loop(0, K, scale, 0)

    pltpu.sync_copy(gb, acc.at[idx_rl], add=True)

  def process_chunks(nch):
    @pl.when(nch > 0)
    def _():
      start_gather(0, 0)

    def pipe(jj, _):
      j0 = jj * 2
      j1 = j0 + 1

      @pl.when(j1 < nch)
      def _():
        start_gather(j1, 1)
      wait_gather(0)
      scale_scatter(j0, 0)

      @pl.when(j1 < nch)
      def _():
        @pl.when(j1 + 1 < nch)
        def _():
          start_gather(j1 + 1, 0)
        wait_gather(1)
        scale_scatter(j1, 1)
      return 0
    lax.fori_loop(0, (nch + 1) // 2, pipe, 0)

  for b in range(BPC):
    lo = (c * BPC + b) * BR

    def zrow(r, _):
      for q in range(DP // L):
        gbuf0[r, pl.ds(q * L, L)] = jnp.zeros((L,), jnp.float32)
      return 0
    lax.fori_loop(0, K, zrow, 0)
    for q in range(6):
      pltpu.sync_copy(gbuf0, acc.at[pl.ds(s * ACC_STRIPE + q * K, K)])
    pltpu.sync_copy(gbuf0.at[pl.ds(0, 16)],
                    acc.at[pl.ds(s * ACC_STRIPE + 6 * K, 16)])
    plsc.subcore_barrier()

    def round_body(r, cnt):
      off = ebase + r * R
      cp_r = pltpu.make_async_copy(row_hbm.at[pl.ds(off, R)], e_row, sem2)
      cp_c = pltpu.make_async_copy(col_hbm.at[pl.ds(off, R)], e_col, sem2)
      cp_w = pltpu.make_async_copy(w_hbm.at[pl.ds(off, R)], e_w, sem2)
      cp_r.start(); cp_c.start(); cp_w.start()
      cp_r.wait(); cp_c.wait(); cp_w.wait()

      def comp(i, cnt):
        rows = e_row[pl.ds(i * L, L)]
        cols = e_col[pl.ds(i * L, L)]
        ws = e_w[pl.ds(i * L, L)]
        m = (rows >= lo) & (rows < lo + BR)
        mi = jnp.where(m, jnp.ones((L,), jnp.int32),
                       jnp.zeros((L,), jnp.int32))
        pos = cnt + plsc.cumsum(mi) - 1
        plsc.store_scatter(b_col, [pos], cols, mask=m)
        plsc.store_scatter(b_w, [pos], ws, mask=m)
        plsc.store_scatter(b_rl, [pos], rows - lo, mask=m)
        return cnt + jnp.sum(mi)
      cnt = lax.fori_loop(0, R // L, comp, cnt)

      nch = cnt // K
      process_chunks(nch)
      rem_base = nch * K
      for q in range(K // L):
        b_col[pl.ds(q * L, L)] = b_col[pl.ds(rem_base + q * L, L)]
        b_w[pl.ds(q * L, L)] = b_w[pl.ds(rem_base + q * L, L)]
        b_rl[pl.ds(q * L, L)] = b_rl[pl.ds(rem_base + q * L, L)]
      return cnt - rem_base
    cnt = lax.fori_loop(0, NR, round_body, jnp.int32(0))

    for q in range(K // L):
      padpos = cnt + q * L + ziota
      plsc.store_scatter(b_col, [padpos], jnp.zeros((L,), jnp.int32))
      plsc.store_scatter(b_w, [padpos], jnp.zeros((L,), jnp.float32))
      plsc.store_scatter(b_rl, [padpos],
                         jnp.full((L,), DUMMY_ROW, jnp.int32))
    process_chunks((cnt + (K - 1)) // K)
    plsc.subcore_barrier()

    last = BR - (NS - 1) * ACC_STRIPE

    @pl.when(s < NS - 1)
    def _():
      pltpu.sync_copy(acc.at[pl.ds(s * ACC_STRIPE, ACC_STRIPE)],
                      out_hbm.at[pl.ds(lo + s * ACC_STRIPE, ACC_STRIPE)])

    @pl.when(s == NS - 1)
    def _():
      pltpu.sync_copy(acc.at[pl.ds((NS - 1) * ACC_STRIPE, last)],
                      out_hbm.at[pl.ds(lo + (NS - 1) * ACC_STRIPE, last)])


_sc_layer = pl.kernel(
    _layer_body,
    out_type=jax.ShapeDtypeStruct((NP, DP), jnp.float32),
    mesh=plsc.VectorSubcoreMesh(core_axis_name="c", subcore_axis_name="s",
                                num_cores=NC, num_subcores=NS),
    compiler_params=pltpu.CompilerParams(needs_layout_passes=False,
                                         use_tc_tiling_on_sc=False),
    scratch_types=[
        pltpu.VMEM((R,), jnp.int32),
        pltpu.VMEM((R,), jnp.int32),
        pltpu.VMEM((R,), jnp.float32),
        pltpu.VMEM((BCAP,), jnp.int32),
        pltpu.VMEM((BCAP,), jnp.float32),
        pltpu.VMEM((BCAP,), jnp.int32),
        pltpu.VMEM((K,), jnp.int32),
        pltpu.VMEM((K, DP), jnp.float32),
        pltpu.VMEM((K, DP), jnp.float32),
        pltpu.VMEM_SHARED((ACC_ROWS, DP), jnp.float32),
        pltpu.SemaphoreType.DMA,
        pltpu.SemaphoreType.DMA,
        pltpu.SemaphoreType.DMA,
    ],
)


_CROWS = 3128


def _combine_body(a_ref, h0, h1, h2, h3, o_ref):
  acc = jnp.zeros((_CROWS, DP), jnp.float32)
  for l, h in enumerate((h0, h1, h2, h3)):
    x = h[...]
    ss = jnp.sum(x * x, axis=-1, keepdims=True)
    nrm = jnp.maximum(jnp.sqrt(ss), 1e-12)
    acc = acc + a_ref[l] * (x / nrm)
  o_ref[...] = acc


_combine = pl.pallas_call(
    _combine_body,
    grid=(NP // _CROWS,),
    in_specs=[
        pl.BlockSpec(memory_space=pltpu.SMEM),
    ] + [pl.BlockSpec((_CROWS, DP), lambda i: (i, 0)) for _ in range(4)],
    out_specs=pl.BlockSpec((_CROWS, DP), lambda i: (i, 0)),
    out_shape=jax.ShapeDtypeStruct((NP, DP), jnp.float32),
)


def kernel(edge_index, edge_weight, embedding, a):
  row = edge_index[0]
  col = edge_index[1]
  x0 = jnp.pad(embedding, ((0, NP - N), (0, DP - D)))
  h1 = _sc_layer(row, col, edge_weight, x0)
  h2 = _sc_layer(row, col, edge_weight, h1)
  h3 = _sc_layer(row, col, edge_weight, h2)
  out = _combine(a.reshape(-1), x0, h1, h2, h3)
  return out[:N, :D]
